# Initial kernel scaffold; baseline (speedup 1.0000x reference)
import jax
import jax.numpy as jnp
from jax.experimental import pallas as pl


def kernel(x, edge_index, params):
    raise NotImplementedError("write your pallas kernel here")


# speedup vs baseline: 27.8634x; 27.8634x over previous
import functools

import jax
import jax.numpy as jnp
from jax import lax
from jax.experimental import pallas as pl
from jax.experimental.pallas import tpu as pltpu
from jax.experimental.pallas import tpu_sc as plsc

N = 10000
E = 320000
LEVELS = 10
B = 1000
FEAT = 128
HID = 128

NLV = LEVELS - 1
WPL = 10
WROWS = B // WPL
WWORDS = WROWS * B
NU = NLV * WPL
NW = 32
CAP = 8192
HPAD = 100096
NB = 112
KEY_SHIFT = 14


def _sc_build_counts(ekey_hbm, bounds_hbm, c_hbm, bounds_v, stage_v, hist_v):
    cid = lax.axis_index("c")
    sid = lax.axis_index("s")
    wid = sid * 2 + cid
    pltpu.sync_copy(bounds_hbm, bounds_v)
    lane = lax.iota(jnp.int32, 16)
    ones = jnp.ones((16,), jnp.float32)
    zeros16 = jnp.zeros((16,), jnp.float32)

    for k in range(3):
        u = wid + k * NW

        @pl.when(u < NU)
        def _unit():
            bv = bounds_v[pl.ds(u, 16)]
            e_lo = bv[0]
            e_hi = bv[1]
            cnt = e_hi - e_lo
            e_lo8 = (e_lo // 8) * 8
            d0 = e_lo - e_lo8
            pltpu.sync_copy(ekey_hbm.at[pl.ds(e_lo8, CAP)], stage_v)

            def _zero(i, carry):
                for t in range(8):
                    hist_v[pl.ds((i * 8 + t) * 16, 16)] = zeros16
                return carry

            lax.fori_loop(0, HPAD // 128, _zero, 0)

            dst_w_lo = (u // WPL + 1) * B + (u % WPL) * WROWS
            src_lo = (u // WPL) * B
            nv = (cnt + 15) // 16

            def _edges(j, carry):
                idx16 = d0 + j * 16 + lane
                keys = plsc.load_gather(stage_v, [idx16])
                dstv = keys >> KEY_SHIFT
                srcv = keys & ((1 << KEY_SHIFT) - 1)
                off = (dstv - dst_w_lo) * B + (srcv - src_lo)
                m = (j * 16 + lane) < cnt
                off = jnp.where(m, off, 0)
                plsc.addupdate_scatter(hist_v, [off], ones, mask=m)
                return carry

            lax.fori_loop(0, nv, _edges, 0)
            pltpu.sync_copy(hist_v.at[pl.ds(0, WWORDS)],
                            c_hbm.at[pl.ds(u * WWORDS, WWORDS)])


def _build_counts(ekey_pad, bounds):
    mesh = plsc.VectorSubcoreMesh(core_axis_name="c", subcore_axis_name="s",
                                  num_cores=2, num_subcores=16)
    return pl.kernel(
        _sc_build_counts,
        out_type=jax.ShapeDtypeStruct((NU * WWORDS,), jnp.float32),
        mesh=mesh,
        scratch_types=[
            pltpu.VMEM((NB,), jnp.int32),
            pltpu.VMEM((CAP,), jnp.int32),
            pltpu.VMEM((HPAD,), jnp.float32),
        ],
        compiler_params=pltpu.CompilerParams(needs_layout_passes=False),
    )(ekey_pad, bounds)


def _tc_levels(x_ref, c_ref, mats_ref, bias_ref, out_ref, emb_ref):
    i = pl.program_id(0)
    relu = lambda v: jnp.maximum(v, 0.0)
    dot = lambda a, bm: jnp.dot(a, bm, preferred_element_type=jnp.float32)
    xb = x_ref[...]
    base = jnp.tanh(dot(xb, mats_ref[0]) + bias_ref[0:1, :])

    @pl.when(i == 0)
    def _():
        out_ref[...] = base
        emb_ref[...] = base

    @pl.when(i > 0)
    def _():
        emb = emb_ref[...]
        msum = dot(c_ref[0], emb)
        a0 = relu(dot(msum, mats_ref[1]) + bias_ref[1:2, :])
        a1 = relu(dot(a0, mats_ref[2]) + bias_ref[2:3, :])
        a2 = relu(dot(a0 + a1, mats_ref[3]) + bias_ref[3:4, :])
        a3 = relu(dot(a1 + a2, mats_ref[4]) + bias_ref[4:5, :])
        e0 = relu(dot(base, mats_ref[5]) + dot(a3, mats_ref[6]) + bias_ref[5:6, :])
        e1 = relu(dot(e0, mats_ref[7]) + bias_ref[6:7, :])
        e2 = relu(dot(e0 + e1, mats_ref[8]) + bias_ref[7:8, :])
        e3 = relu(dot(e1 + e2, mats_ref[9]) + bias_ref[8:9, :])
        out_ref[...] = e3
        emb_ref[...] = e3


def _run_levels(x, c3, mats, bias):
    return pl.pallas_call(
        _tc_levels,
        grid=(LEVELS,),
        in_specs=[
            pl.BlockSpec((B, FEAT), lambda i: (i, 0)),
            pl.BlockSpec((1, B, B), lambda i: (jnp.maximum(i - 1, 0), 0, 0)),
            pl.BlockSpec((10, HID, HID), lambda i: (0, 0, 0)),
            pl.BlockSpec((16, HID), lambda i: (0, 0)),
        ],
        out_specs=pl.BlockSpec((B, HID), lambda i: (i, 0)),
        out_shape=jax.ShapeDtypeStruct((N, HID), jnp.float32),
        scratch_shapes=[pltpu.VMEM((B, HID), jnp.float32)],
    )(x, c3, mats, bias)


def kernel(x, edge_index, params):
    src = edge_index[0].astype(jnp.int32)
    dst = edge_index[1].astype(jnp.int32)
    ekey = jnp.sort((dst << KEY_SHIFT) | src)
    ekey_pad = jnp.concatenate([ekey, jnp.zeros((CAP,), jnp.int32)])
    marks = (B + jnp.arange(NU + 1, dtype=jnp.int32) * WROWS) << KEY_SHIFT
    bounds = jnp.searchsorted(ekey, marks, side="left").astype(jnp.int32)
    bounds = jnp.concatenate(
        [bounds, jnp.full((NB - NU - 1,), E, jnp.int32)])

    c_flat = _build_counts(ekey_pad, bounds)
    c3 = c_flat.reshape(NLV, B, B)

    ne0 = params["ne_w"][0]
    mats = jnp.stack(
        [params["We"]] + list(params["mp_w"])
        + [ne0[:HID], ne0[HID:]] + list(params["ne_w"][1:]))
    bias = jnp.stack(
        [params["be"]] + list(params["mp_b"]) + list(params["ne_b"]))
    bias = jnp.concatenate(
        [bias, jnp.zeros((16 - bias.shape[0], HID), jnp.float32)])

    return _run_levels(x, c3, mats, bias)
